# Initial kernel scaffold; baseline (speedup 1.0000x reference)
import jax
import jax.numpy as jnp
from jax.experimental import pallas as pl


def kernel(x, gate, w1, b1, w2, b2):
    raise NotImplementedError("write your pallas kernel here")


# speedup vs baseline: 1.6102x; 1.6102x over previous
import functools

import jax
import jax.numpy as jnp
from jax.experimental import pallas as pl
from jax.experimental.pallas import tpu as pltpu

_TOPK = 2


def _ff_kernel(tsel_ref, x_ref, smx_ref, b1_ref, b2_ref, w1_ref, w2_ref,
               out_ref, xg_ref, yacc_ref, *, n_chunks, cutoff, dmodel, npairs):
    j = pl.program_id(0)

    @pl.when(j == 0)
    def _init():
        out_ref[:, :] = jnp.broadcast_to(b2_ref[:, :], (cutoff, dmodel))
        yacc_ref[:, :] = jnp.zeros((npairs, dmodel), jnp.float32)
        for p in range(npairs):
            t = tsel_ref[p]
            xg_ref[pl.ds(p, 1), :] = x_ref[pl.ds(t, 1), :]

    h = jnp.dot(xg_ref[:, :], w1_ref[:, :], preferred_element_type=jnp.float32)
    h = jnp.maximum(h + b1_ref[:, :], 0.0)
    g = h * smx_ref[:, :]
    yacc_ref[:, :] += jnp.dot(g, w2_ref[:, :],
                              preferred_element_type=jnp.float32)

    @pl.when(j == n_chunks - 1)
    def _scatter():
        y = yacc_ref[:, :]
        for p in range(npairs):
            t = tsel_ref[p]
            out_ref[pl.ds(t, 1), :] = out_ref[pl.ds(t, 1), :] + y[p:p + 1, :]


def kernel(x, gate, w1, b1, w2, b2):
    b, c, d = x.shape
    E = gate.shape[1] // c
    width = E * d
    npairs = E * _TOPK

    gate_r = gate.reshape(c, c, E)
    gate_out = jnp.einsum('bcd,cke->bkde', x, gate_r)
    gate_out = jax.nn.softmax(gate_out, axis=-2)
    scores = gate_out.mean(axis=2)
    topv, topi = jax.lax.top_k(jnp.swapaxes(scores, 1, 2), _TOPK)

    tsel = topi.reshape(npairs).astype(jnp.int32)
    eflat = jnp.arange(E, dtype=jnp.int32).repeat(_TOPK)
    sel = gate_out[0][tsel]
    sel_sm = jnp.take_along_axis(sel, eflat[:, None, None], axis=2)[:, :, 0]
    onehot = (eflat[:, None] == jnp.arange(E)[None, :]).astype(jnp.float32)
    smx = (sel_sm[:, :, None] * onehot[:, None, :]).reshape(npairs, width)

    n_chunks = E
    chunk = width // n_chunks

    out2d = pl.pallas_call(
        functools.partial(_ff_kernel, n_chunks=n_chunks, cutoff=c, dmodel=d,
                          npairs=npairs),
        grid=(n_chunks,),
        in_specs=[
            pl.BlockSpec(memory_space=pltpu.SMEM),
            pl.BlockSpec((c, d), lambda j: (0, 0)),
            pl.BlockSpec((npairs, chunk), lambda j: (0, j)),
            pl.BlockSpec((1, chunk), lambda j: (0, j)),
            pl.BlockSpec((1, d), lambda j: (0, 0)),
            pl.BlockSpec((d, chunk), lambda j: (0, j)),
            pl.BlockSpec((chunk, d), lambda j: (j, 0)),
        ],
        out_specs=pl.BlockSpec((c, d), lambda j: (0, 0)),
        out_shape=jax.ShapeDtypeStruct((c, d), jnp.float32),
        scratch_shapes=[pltpu.VMEM((npairs, d), jnp.float32),
                        pltpu.VMEM((npairs, d), jnp.float32)],
    )(tsel, x[0], smx, b1.reshape(1, width), b2.reshape(1, d), w1, w2)

    return out2d.reshape(b, c, d)


# speedup vs baseline: 1.6534x; 1.0268x over previous
import functools

import jax
import jax.numpy as jnp
from jax.experimental import pallas as pl
from jax.experimental.pallas import tpu as pltpu

_TOPK = 2


def _ff_kernel(tsel_ref, x_ref, smx_ref, b1_ref, b2_ref, w1_ref, w2_ref,
               out_ref, xg_ref, yacc_ref, *, n_chunks, cutoff, dmodel, npairs):
    j = pl.program_id(0)

    @pl.when(j == 0)
    def _init():
        out_ref[:, :] = jnp.broadcast_to(b2_ref[:, :], (cutoff, dmodel))
        yacc_ref[:, :] = jnp.zeros((npairs, dmodel), jnp.float32)
        for p in range(npairs):
            t = tsel_ref[p]
            xg_ref[pl.ds(p, 1), :] = x_ref[pl.ds(t, 1), :]

    h = jnp.dot(xg_ref[:, :], w1_ref[:, :], preferred_element_type=jnp.float32)
    h = jnp.maximum(h + b1_ref[:, :], 0.0)
    g = h * smx_ref[:, :]
    yacc_ref[:, :] += jnp.dot(g, w2_ref[:, :],
                              preferred_element_type=jnp.float32)

    @pl.when(j == n_chunks - 1)
    def _scatter():
        y = yacc_ref[:, :]
        for p in range(npairs):
            t = tsel_ref[p]
            out_ref[pl.ds(t, 1), :] = out_ref[pl.ds(t, 1), :] + y[p:p + 1, :]


def kernel(x, gate, w1, b1, w2, b2):
    b, c, d = x.shape
    E = gate.shape[1] // c
    width = E * d
    npairs = E * _TOPK

    gate_r = gate.reshape(c, c, E)
    gate_out = jnp.einsum('bcd,cke->bkde', x, gate_r)
    sm = jax.nn.softmax(gate_out, axis=-2)
    scores = sm.mean(axis=2)
    scores2 = jnp.swapaxes(scores, 1, 2)
    i1 = jnp.argmax(scores2, axis=2).astype(jnp.int32)
    iota = jax.lax.broadcasted_iota(jnp.int32, scores2.shape, 2)
    masked = jnp.where(iota == i1[:, :, None], -jnp.inf, scores2)
    i2 = jnp.argmax(masked, axis=2).astype(jnp.int32)
    topi = jnp.stack([i1, i2], axis=-1)

    tsel = topi.reshape(npairs).astype(jnp.int32)
    eflat = jnp.arange(E, dtype=jnp.int32).repeat(_TOPK)
    sel_g = gate_out[0][tsel]
    sel_g = jnp.take_along_axis(sel_g, eflat[:, None, None], axis=2)[:, :, 0]
    sel_sm = jax.nn.softmax(sel_g, axis=-1)
    onehot = (eflat[:, None] == jnp.arange(E)[None, :]).astype(jnp.float32)
    smx = (sel_sm[:, :, None] * onehot[:, None, :]).reshape(npairs, width)

    n_chunks = E
    chunk = width // n_chunks

    out2d = pl.pallas_call(
        functools.partial(_ff_kernel, n_chunks=n_chunks, cutoff=c, dmodel=d,
                          npairs=npairs),
        grid=(n_chunks,),
        in_specs=[
            pl.BlockSpec(memory_space=pltpu.SMEM),
            pl.BlockSpec((c, d), lambda j: (0, 0)),
            pl.BlockSpec((npairs, chunk), lambda j: (0, j)),
            pl.BlockSpec((1, chunk), lambda j: (0, j)),
            pl.BlockSpec((1, d), lambda j: (0, 0)),
            pl.BlockSpec((d, chunk), lambda j: (0, j)),
            pl.BlockSpec((chunk, d), lambda j: (j, 0)),
        ],
        out_specs=pl.BlockSpec((c, d), lambda j: (0, 0)),
        out_shape=jax.ShapeDtypeStruct((c, d), jnp.float32),
        scratch_shapes=[pltpu.VMEM((npairs, d), jnp.float32),
                        pltpu.VMEM((npairs, d), jnp.float32)],
    )(tsel, x[0], smx, b1.reshape(1, width), b2.reshape(1, d), w1, w2)

    return out2d.reshape(b, c, d)
